# baseline (speedup 1.0000x reference)
import jax
import jax.numpy as jnp
from jax.experimental import pallas as pl
from jax.experimental.pallas import tpu as pltpu

_D_MODEL = 768
_D_HID = 384
_N_EXP = 64
_BLK = 4096


def _router_body(x_ref, emb_ref, w1_ref, b1_ref, w2_ref, b2_ref, ones_ref,
                 w_out_ref, idx_out_ref):
    xc = x_ref[...] + emb_ref[...]
    h = jnp.dot(xc, w1_ref[...], preferred_element_type=jnp.float32)
    h = jnp.maximum(h + b1_ref[...], 0.0)
    logits = jnp.dot(h, w2_ref[...], preferred_element_type=jnp.float32)
    logits = logits + b2_ref[...]

    m = jnp.max(logits, axis=-1, keepdims=True)
    e = jnp.exp(logits - m)
    s = jnp.dot(e, ones_ref[...], preferred_element_type=jnp.float32)
    w_out_ref[...] = e / s

    idx = jax.lax.broadcasted_iota(jnp.int32, e.shape, 1)
    big = jnp.int32(_N_EXP)
    i1 = jnp.min(jnp.where(e == 1.0, idx, big), axis=-1, keepdims=True)
    e2 = jnp.where(idx == i1, jnp.float32(0.0), e)
    m2 = jnp.max(e2, axis=-1, keepdims=True)
    i2 = jnp.min(jnp.where(e2 == m2, idx, big), axis=-1, keepdims=True)
    idx_out_ref[...] = jnp.concatenate([i1, i2], axis=-1)


def kernel(x, table, W1, b1, W2, b2):
    batch, seq, d_model = x.shape
    n_tok = batch * seq
    x2 = x.reshape(n_tok, d_model)
    emb = table[0].reshape(1, d_model)
    b1r = b1.reshape(1, _D_HID)
    b2r = b2.reshape(1, _N_EXP)
    ones = jnp.ones((_N_EXP, _N_EXP), jnp.float32)

    grid = (n_tok // _BLK,)
    weights, indices = pl.pallas_call(
        _router_body,
        grid=grid,
        in_specs=[
            pl.BlockSpec((_BLK, d_model), lambda i: (i, 0)),
            pl.BlockSpec((1, d_model), lambda i: (0, 0)),
            pl.BlockSpec((d_model, _D_HID), lambda i: (0, 0)),
            pl.BlockSpec((1, _D_HID), lambda i: (0, 0)),
            pl.BlockSpec((_D_HID, _N_EXP), lambda i: (0, 0)),
            pl.BlockSpec((1, _N_EXP), lambda i: (0, 0)),
            pl.BlockSpec((_N_EXP, _N_EXP), lambda i: (0, 0)),
        ],
        out_specs=[
            pl.BlockSpec((_BLK, _N_EXP), lambda i: (i, 0)),
            pl.BlockSpec((_BLK, 2), lambda i: (i, 0)),
        ],
        out_shape=[
            jax.ShapeDtypeStruct((n_tok, _N_EXP), jnp.float32),
            jax.ShapeDtypeStruct((n_tok, 2), jnp.int32),
        ],
        compiler_params=pltpu.CompilerParams(
            dimension_semantics=("parallel",)),
    )(x2, emb, W1, b1r, W2, b2r, ones)

    return (weights.reshape(batch, seq, _N_EXP),
            indices.reshape(batch, seq, 2))


# speedup vs baseline: 1.0644x; 1.0644x over previous
import jax
import jax.numpy as jnp
from jax.experimental import pallas as pl

_D_MODEL = 768
_D_HID = 384
_N_EXP = 64
_BLK = 4096


def _router_body(x_ref, emb_ref, w1_ref, b1_ref, w2_ref, b2_ref, ones_ref,
                 w_out_ref, idx_out_ref):
    xc = x_ref[...] + emb_ref[...]
    h = jnp.dot(xc, w1_ref[...], preferred_element_type=jnp.float32)
    h = jnp.maximum(h + b1_ref[...], 0.0)
    logits = jnp.dot(h, w2_ref[...], preferred_element_type=jnp.float32)
    logits = logits + b2_ref[...]

    m = jnp.max(logits, axis=-1, keepdims=True)
    e = jnp.exp(logits - m)
    s = jnp.dot(e, ones_ref[...], preferred_element_type=jnp.float32)
    w_out_ref[...] = e / s

    idx = jax.lax.broadcasted_iota(jnp.int32, e.shape, 1)
    big = jnp.int32(_N_EXP)
    i1 = jnp.min(jnp.where(e == 1.0, idx, big), axis=-1, keepdims=True)
    e2 = jnp.where(idx == i1, jnp.float32(0.0), e)
    m2 = jnp.max(e2, axis=-1, keepdims=True)
    i2 = jnp.min(jnp.where(e2 == m2, idx, big), axis=-1, keepdims=True)
    idx_out_ref[...] = jnp.concatenate([i1, i2], axis=-1)


def kernel(x, table, W1, b1, W2, b2):
    batch, seq, d_model = x.shape
    emb = table[0].reshape(1, d_model)
    b1r = b1.reshape(1, _D_HID)
    b2r = b2.reshape(1, _N_EXP)
    ones = jnp.ones((_N_EXP, _N_EXP), jnp.float32)

    grid = (batch, seq // _BLK)
    weights, indices = pl.pallas_call(
        _router_body,
        grid=grid,
        in_specs=[
            pl.BlockSpec((None, _BLK, d_model), lambda b, i: (b, i, 0)),
            pl.BlockSpec((1, d_model), lambda b, i: (0, 0)),
            pl.BlockSpec((d_model, _D_HID), lambda b, i: (0, 0)),
            pl.BlockSpec((1, _D_HID), lambda b, i: (0, 0)),
            pl.BlockSpec((_D_HID, _N_EXP), lambda b, i: (0, 0)),
            pl.BlockSpec((1, _N_EXP), lambda b, i: (0, 0)),
            pl.BlockSpec((_N_EXP, _N_EXP), lambda b, i: (0, 0)),
        ],
        out_specs=[
            pl.BlockSpec((None, _BLK, _N_EXP), lambda b, i: (b, i, 0)),
            pl.BlockSpec((None, _BLK, 2), lambda b, i: (b, i, 0)),
        ],
        out_shape=[
            jax.ShapeDtypeStruct((batch, seq, _N_EXP), jnp.float32),
            jax.ShapeDtypeStruct((batch, seq, 2), jnp.int32),
        ],
    )(x, emb, W1, b1r, W2, b2r, ones)

    return weights, indices


# speedup vs baseline: 1.3864x; 1.3026x over previous
import jax
import jax.numpy as jnp
from jax.experimental import pallas as pl

_D_MODEL = 768
_D_HID = 384
_N_EXP = 64
_BLK = 4096


_ABT = (((1,), (1,)), ((), ()))


def _router_body(x_ref, emb_ref, w1_ref, b1_ref, w2_ref, b2_ref, mt_ref,
                 i2t_ref, w_out_ref, idx_out_ref):
    xc = x_ref[...] + emb_ref[...]
    h = jnp.dot(xc, w1_ref[...], preferred_element_type=jnp.float32)
    h = jnp.maximum(h + b1_ref[...], 0.0)
    logits = jnp.dot(h, w2_ref[...], preferred_element_type=jnp.float32)
    logits = logits + b2_ref[...]

    m = jnp.max(logits, axis=-1, keepdims=True)
    e = jnp.exp(logits - m)

    t = jax.lax.dot_general(mt_ref[...], e, _ABT,
                            preferred_element_type=jnp.float32)
    w_out_ref[...] = t[0:_N_EXP, :] / t[_N_EXP:_N_EXP + 1, :]

    idx = jax.lax.broadcasted_iota(jnp.int32, e.shape, 1)
    big = jnp.int32(_N_EXP)
    i1 = jnp.min(jnp.where(e == 1.0, idx, big), axis=-1, keepdims=True)
    e2 = jnp.where(idx == i1, jnp.float32(0.0), e)
    m2 = jnp.max(e2, axis=-1, keepdims=True)
    i2 = jnp.min(jnp.where(e2 == m2, idx, big), axis=-1, keepdims=True)
    iv = jnp.concatenate([i1, i2], axis=-1).astype(jnp.float32)
    it = jax.lax.dot_general(i2t_ref[...], iv, _ABT,
                             preferred_element_type=jnp.float32)
    idx_out_ref[...] = it[0:2, :].astype(jnp.int32)


def kernel(x, table, W1, b1, W2, b2):
    batch, seq, d_model = x.shape
    emb = table[0].reshape(1, d_model)
    b1r = b1.reshape(1, _D_HID)
    b2r = b2.reshape(1, _N_EXP)
    mt = jnp.concatenate([jnp.eye(_N_EXP, dtype=jnp.float32),
                          jnp.ones((1, _N_EXP), jnp.float32),
                          jnp.zeros((7, _N_EXP), jnp.float32)], axis=0)
    i2t = jnp.concatenate([jnp.eye(2, dtype=jnp.float32),
                           jnp.zeros((6, 2), jnp.float32)], axis=0)

    grid = (batch, seq // _BLK)
    weights, indices = pl.pallas_call(
        _router_body,
        grid=grid,
        in_specs=[
            pl.BlockSpec((None, _BLK, d_model), lambda b, i: (b, i, 0)),
            pl.BlockSpec((1, d_model), lambda b, i: (0, 0)),
            pl.BlockSpec((d_model, _D_HID), lambda b, i: (0, 0)),
            pl.BlockSpec((1, _D_HID), lambda b, i: (0, 0)),
            pl.BlockSpec((_D_HID, _N_EXP), lambda b, i: (0, 0)),
            pl.BlockSpec((1, _N_EXP), lambda b, i: (0, 0)),
            pl.BlockSpec((_N_EXP + 8, _N_EXP), lambda b, i: (0, 0)),
            pl.BlockSpec((8, 2), lambda b, i: (0, 0)),
        ],
        out_specs=[
            pl.BlockSpec((None, _N_EXP, _BLK), lambda b, i: (b, 0, i)),
            pl.BlockSpec((None, 2, _BLK), lambda b, i: (b, 0, i)),
        ],
        out_shape=[
            jax.ShapeDtypeStruct((batch, _N_EXP, seq), jnp.float32),
            jax.ShapeDtypeStruct((batch, 2, seq), jnp.int32),
        ],
    )(x, emb, W1, b1r, W2, b2r, mt, i2t)

    return jnp.swapaxes(weights, 1, 2), jnp.swapaxes(indices, 1, 2)
